# baseline (speedup 1.0000x reference)
import functools

import jax
import jax.numpy as jnp
from jax import lax
from jax.experimental import pallas as pl
from jax.experimental.pallas import tpu as pltpu
from jax.experimental.pallas import tpu_sc as plsc

_B = 16384
_D = 64
_V = 100001
_L = 16

_info = plsc.get_sparse_core_info()
_NC = _info.num_cores
_NS = _info.num_subcores
_NW = _NC * _NS
_DPW = _D // _NW
_OC = 4096
_NOC = _B // _OC
_UNROLL = 8


def _gather_body(idx_hbm, tab_t_hbm, out_t_hbm, idx_v, row_v, out_v, rsem, osem):
    wid = lax.axis_index("s") * _NC + lax.axis_index("c")

    row0 = pltpu.make_async_copy(tab_t_hbm.at[wid * _DPW], row_v, rsem)
    row0.start()
    pltpu.sync_copy(idx_hbm, idx_v)
    row0.wait()

    for fd in range(_DPW):
        d = wid * _DPW + fd
        for c in range(_NOC):
            buf = c % 2

            def gbody(j, _):
                for u in range(_UNROLL):
                    k = (j * _UNROLL + u) * _L
                    sl_out = pl.ds(k, _L)
                    sl_idx = pl.ds(c * _OC + k, _L)
                    out_v[buf, sl_out] = plsc.load_gather(
                        row_v, [idx_v[sl_idx] + 1])
                return _

            if c >= 2:
                pltpu.make_async_copy(
                    out_v.at[buf],
                    out_t_hbm.at[d, pl.ds((c - 2) * _OC, _OC)],
                    osem).wait()
            lax.fori_loop(0, _OC // (_L * _UNROLL), gbody, 0)
            pltpu.make_async_copy(
                out_v.at[buf], out_t_hbm.at[d, pl.ds(c * _OC, _OC)],
                osem).start()

        for c in range(_NOC - 2, _NOC):
            pltpu.make_async_copy(
                out_v.at[c % 2], out_t_hbm.at[d, pl.ds(c * _OC, _OC)],
                osem).wait()
        if fd + 1 < _DPW:
            pltpu.sync_copy(tab_t_hbm.at[d + 1], row_v)


@jax.jit
def kernel(style_idx, embeddings):
    mesh = plsc.VectorSubcoreMesh(core_axis_name="c", subcore_axis_name="s")
    f = functools.partial(
        pl.kernel,
        mesh=mesh,
        out_type=jax.ShapeDtypeStruct((_D, _B), jnp.float32),
        compiler_params=pltpu.CompilerParams(needs_layout_passes=False),
        scratch_types=[
            pltpu.VMEM((_B,), jnp.int32),
            pltpu.VMEM((_V,), jnp.float32),
            pltpu.VMEM((2, _OC), jnp.float32),
            pltpu.SemaphoreType.DMA,
            pltpu.SemaphoreType.DMA,
        ],
    )(_gather_body)
    out_t = f(style_idx, embeddings.T)
    return out_t.T


# speedup vs baseline: 1.1234x; 1.1234x over previous
import functools

import jax
import jax.numpy as jnp
from jax import lax
from jax.experimental import pallas as pl
from jax.experimental.pallas import tpu as pltpu
from jax.experimental.pallas import tpu_sc as plsc

_B = 16384
_D = 64
_V = 100001
_L = 16

_info = plsc.get_sparse_core_info()
_NC = _info.num_cores
_NS = _info.num_subcores
_NW = _NC * _NS
_DPW = _D // _NW
_OC = 8192
_NOC = _B // _OC
_UNROLL = 4


def _gather_body(idx_hbm, tab_t_hbm, out_t_hbm, idx_v, row_v, out_v, rsem, osem):
    wid = lax.axis_index("s") * _NC + lax.axis_index("c")

    row0 = pltpu.make_async_copy(tab_t_hbm.at[wid * _DPW], row_v, rsem)
    row0.start()
    pltpu.sync_copy(idx_hbm, idx_v)
    row0.wait()

    for fd in range(_DPW):
        d = wid * _DPW + fd
        for c in range(_NOC):

            def gbody(j, _):
                for u in range(_UNROLL):
                    k = (j * _UNROLL + u) * _L
                    out_v[pl.ds(k, _L)] = plsc.load_gather(
                        row_v, [idx_v[pl.ds(c * _OC + k, _L)] + 1])
                return _

            lax.fori_loop(0, _OC // (_L * _UNROLL), gbody, 0)
            pltpu.sync_copy(out_v, out_t_hbm.at[d, pl.ds(c * _OC, _OC)])
        if fd + 1 < _DPW:
            pltpu.sync_copy(tab_t_hbm.at[d + 1], row_v)


@jax.jit
def kernel(style_idx, embeddings):
    mesh = plsc.VectorSubcoreMesh(core_axis_name="c", subcore_axis_name="s")
    f = functools.partial(
        pl.kernel,
        mesh=mesh,
        out_type=jax.ShapeDtypeStruct((_D, _B), jnp.float32),
        compiler_params=pltpu.CompilerParams(needs_layout_passes=False),
        scratch_types=[
            pltpu.VMEM((_B,), jnp.int32),
            pltpu.VMEM((_V,), jnp.float32),
            pltpu.VMEM((_OC,), jnp.float32),
            pltpu.SemaphoreType.DMA,
            pltpu.SemaphoreType.DMA,
        ],
    )(_gather_body)
    out_t = f(style_idx, embeddings.T)
    return out_t.T


# speedup vs baseline: 1.1657x; 1.0376x over previous
import functools

import jax
import jax.numpy as jnp
from jax import lax
from jax.experimental import pallas as pl
from jax.experimental.pallas import tpu as pltpu
from jax.experimental.pallas import tpu_sc as plsc

_B = 16384
_D = 64
_V = 100001
_L = 16

_info = plsc.get_sparse_core_info()
_NC = _info.num_cores
_NS = _info.num_subcores
_NW = _NC * _NS
_DPW = _D // _NW
_IC = 8192
_NIC = _B // _IC
_UNROLL = 4


def _gather_body(idx_hbm, tab_t_hbm, out_t_hbm, idx_v, row_v, out_v):
    wid = lax.axis_index("s") * _NC + lax.axis_index("c")
    for fd in range(_DPW):
        d = wid * _DPW + fd
        pltpu.sync_copy(tab_t_hbm.at[d], row_v)
        for c in range(_NIC):
            pltpu.sync_copy(idx_hbm.at[pl.ds(c * _IC, _IC)], idx_v)

            def gbody(j, _):
                for u in range(_UNROLL):
                    sl = pl.ds((j * _UNROLL + u) * _L, _L)
                    out_v[sl] = plsc.load_gather(row_v, [idx_v[sl] + 1])
                return _

            lax.fori_loop(0, _IC // (_L * _UNROLL), gbody, 0)
            pltpu.sync_copy(out_v, out_t_hbm.at[d, pl.ds(c * _IC, _IC)])


@jax.jit
def kernel(style_idx, embeddings):
    mesh = plsc.VectorSubcoreMesh(core_axis_name="c", subcore_axis_name="s")
    f = functools.partial(
        pl.kernel,
        mesh=mesh,
        out_type=jax.ShapeDtypeStruct((_D, _B), jnp.float32),
        compiler_params=pltpu.CompilerParams(
            needs_layout_passes=False, skip_device_barrier=True),
        scratch_types=[
            pltpu.VMEM((_IC,), jnp.int32),
            pltpu.VMEM((_V,), jnp.float32),
            pltpu.VMEM((_IC,), jnp.float32),
        ],
    )(_gather_body)
    out_t = f(style_idx, embeddings.T)
    return out_t.T
